# baseline (speedup 1.0000x reference)
import functools

import jax
import jax.numpy as jnp
from jax import lax
from jax.experimental import pallas as pl
from jax.experimental.pallas import tpu as pltpu
from jax.experimental.pallas import tpu_sc as plsc

_L = 16

_MESH = plsc.VectorSubcoreMesh(core_axis_name="c", subcore_axis_name="s")


@functools.partial(
    pl.kernel,
    mesh=_MESH,
    out_type=jax.ShapeDtypeStruct((32, 96), jnp.float32),
    scratch_types=[
        pltpu.VMEM((64,), jnp.float32),
        pltpu.VMEM((64,), jnp.int32),
        pltpu.VMEM((32,), jnp.float32),
        pltpu.VMEM((96,), jnp.float32),
    ],
)
def _sc_segsum_outer(val_hbm, idx_hbm, w_hbm, out_hbm, val_v, idx_v, w_v, row_v):
    c = lax.axis_index("c")
    s = lax.axis_index("s")
    wid = s * 2 + c

    zeros = jnp.zeros((_L,), jnp.float32)
    val_v[pl.ds(48, _L)] = zeros
    pltpu.sync_copy(val_hbm, val_v.at[pl.ds(0, 50)])
    pltpu.sync_copy(idx_hbm, idx_v.at[pl.ds(0, 50)])
    pltpu.sync_copy(w_hbm, w_v)

    acc = zeros
    for k in range(4):
        v = val_v[pl.ds(k * _L, _L)]
        ix = idx_v[pl.ds(k * _L, _L)]
        acc = acc + jnp.where(ix == wid, v, zeros)
    seg = acc[0]
    for i in range(1, _L):
        seg = seg + acc[i]

    row_v[pl.ds(0, _L)] = seg * w_v[pl.ds(0, _L)]
    row_v[pl.ds(_L, _L)] = seg * w_v[pl.ds(_L, _L)]
    for k in range(2, 6):
        row_v[pl.ds(k * _L, _L)] = zeros
    pltpu.sync_copy(row_v, out_hbm.at[wid])


def kernel(arg1_1, arg2_1, W):
    return _sc_segsum_outer(arg1_1.reshape(50), arg2_1, W)


# speedup vs baseline: 1.0775x; 1.0775x over previous
import functools

import jax
import jax.numpy as jnp
from jax import lax
from jax.experimental import pallas as pl
from jax.experimental.pallas import tpu as pltpu
from jax.experimental.pallas import tpu_sc as plsc

_L = 16

_MESH = plsc.VectorSubcoreMesh(core_axis_name="c", subcore_axis_name="s", num_cores=1)


@functools.partial(
    pl.kernel,
    mesh=_MESH,
    out_type=jax.ShapeDtypeStruct((32, 96), jnp.float32),
    scratch_types=[
        pltpu.VMEM((64,), jnp.float32),
        pltpu.VMEM((64,), jnp.int32),
        pltpu.VMEM((32,), jnp.float32),
        pltpu.VMEM((96,), jnp.float32),
    ],
)
def _sc_segsum_outer(val_hbm, idx_hbm, w_hbm, out_hbm, val_v, idx_v, w_v, row_v):
    s = lax.axis_index("s")

    zeros = jnp.zeros((_L,), jnp.float32)
    val_v[pl.ds(48, _L)] = zeros
    pltpu.sync_copy(val_hbm, val_v.at[pl.ds(0, 50)])
    pltpu.sync_copy(idx_hbm, idx_v.at[pl.ds(0, 50)])
    pltpu.sync_copy(w_hbm, w_v)

    vals = [val_v[pl.ds(k * _L, _L)] for k in range(4)]
    ixs = [idx_v[pl.ds(k * _L, _L)] for k in range(4)]
    w0 = w_v[pl.ds(0, _L)]
    w1 = w_v[pl.ds(_L, _L)]
    for r in range(2):
        wid = s * 2 + r
        acc = zeros
        for k in range(4):
            acc = acc + jnp.where(ixs[k] == wid, vals[k], zeros)
        seg = acc[0]
        for i in range(1, _L):
            seg = seg + acc[i]
        row_v[pl.ds(0, _L)] = seg * w0
        row_v[pl.ds(_L, _L)] = seg * w1
        for k in range(2, 6):
            row_v[pl.ds(k * _L, _L)] = zeros
        pltpu.sync_copy(row_v, out_hbm.at[wid])


def kernel(arg1_1, arg2_1, W):
    return _sc_segsum_outer(arg1_1.reshape(50), arg2_1, W)


# speedup vs baseline: 1.1451x; 1.0627x over previous
import functools

import jax
import jax.numpy as jnp
from jax import lax
from jax.experimental import pallas as pl
from jax.experimental.pallas import tpu as pltpu
from jax.experimental.pallas import tpu_sc as plsc

_L = 16

_MESH = plsc.VectorSubcoreMesh(core_axis_name="c", subcore_axis_name="s", num_cores=1)


@functools.partial(
    pl.kernel,
    mesh=_MESH,
    out_type=jax.ShapeDtypeStruct((32, 96), jnp.float32),
    scratch_types=[
        pltpu.VMEM((64,), jnp.float32),
        pltpu.VMEM((64,), jnp.int32),
        pltpu.VMEM((32,), jnp.float32),
        pltpu.VMEM((2, 96), jnp.float32),
        pltpu.SemaphoreType.DMA,
    ],
)
def _sc_segsum_outer(val_hbm, idx_hbm, w_hbm, out_hbm, val_v, idx_v, w_v, rows_v, sem):
    s = lax.axis_index("s")

    c1 = pltpu.async_copy(val_hbm, val_v.at[pl.ds(0, 50)], sem)
    c2 = pltpu.async_copy(idx_hbm, idx_v.at[pl.ds(0, 50)], sem)
    c3 = pltpu.async_copy(w_hbm, w_v, sem)
    c1.wait()
    c2.wait()
    c3.wait()

    zeros = jnp.zeros((_L,), jnp.float32)
    lane = lax.iota(jnp.int32, _L)
    vals = [val_v[pl.ds(k * _L, _L)] for k in range(4)]
    ixs = [idx_v[pl.ds(k * _L, _L)] for k in range(4)]
    w0 = w_v[pl.ds(0, _L)]
    w1 = w_v[pl.ds(_L, _L)]
    for r in range(2):
        wid = s * 2 + r
        acc = zeros
        for k in range(4):
            m = ixs[k] == wid
            if k == 3:
                m = m & (lane < 2)
            acc = acc + jnp.where(m, vals[k], zeros)
        seg = acc[0]
        for i in range(1, _L):
            seg = seg + acc[i]
        rows_v[r, pl.ds(0, _L)] = seg * w0
        rows_v[r, pl.ds(_L, _L)] = seg * w1
        for k in range(2, 6):
            rows_v[r, pl.ds(k * _L, _L)] = zeros
    pltpu.sync_copy(rows_v, out_hbm.at[pl.ds(s * 2, 2)])


def kernel(arg1_1, arg2_1, W):
    return _sc_segsum_outer(arg1_1.reshape(50), arg2_1, W)
